# baseline (speedup 1.0000x reference)
import functools

import jax
import jax.numpy as jnp
from jax import lax
from jax.experimental import pallas as pl
from jax.experimental.pallas import tpu as pltpu
from jax.experimental.pallas import tpu_sc as plsc

_B, _N, _C = 8, 4096, 64
_S = 1024
_K = 32
_TS = 128
_TM = 128


def _fps_body(xyzT_ref, nrmT_ref, out_ref):
    lane = lax.broadcasted_iota(jnp.int32, (_B, _N), 1)

    def step(t, carry):
        dist0, nxt = carry
        msk = lane == nxt
        X = xyzT_ref[:, 0, :]
        Y = xyzT_ref[:, 1, :]
        Z = xyzT_ref[:, 2, :]
        px = jnp.sum(jnp.where(msk, X, 0.0), axis=1, keepdims=True)
        py = jnp.sum(jnp.where(msk, Y, 0.0), axis=1, keepdims=True)
        pz = jnp.sum(jnp.where(msk, Z, 0.0), axis=1, keepdims=True)
        nx = jnp.sum(jnp.where(msk, nrmT_ref[:, 0, :], 0.0), axis=1, keepdims=True)
        ny = jnp.sum(jnp.where(msk, nrmT_ref[:, 1, :], 0.0), axis=1, keepdims=True)
        nz = jnp.sum(jnp.where(msk, nrmT_ref[:, 2, :], 0.0), axis=1, keepdims=True)
        row = jnp.concatenate([px, py, pz, nx, ny, nz], axis=1)
        out_ref[pl.ds(t, 1)] = row[None]
        d = (X - px) ** 2 + (Y - py) ** 2 + (Z - pz) ** 2
        dist = jnp.minimum(dist0, d)
        m = jnp.max(dist, axis=1, keepdims=True)
        nxt2 = jnp.min(jnp.where(dist == m, lane, _N), axis=1, keepdims=True)
        return dist, nxt2

    lax.fori_loop(0, _S, step,
                  (jnp.full((_B, _N), 1e10, dtype=jnp.float32),
                   jnp.zeros((_B, 1), jnp.int32)))


def _run_fps(xyzT, nrmT):
    return pl.pallas_call(
        _fps_body,
        out_shape=jax.ShapeDtypeStruct((_S, _B, 6), jnp.float32),
    )(xyzT, nrmT)


def _g_body(pf_ref, w_ref, out_ref):
    out_ref[0] = jnp.dot(pf_ref[0], w_ref[...],
                         preferred_element_type=jnp.float32)


def _run_g(pf, w1a):
    return pl.pallas_call(
        _g_body,
        grid=(_B,),
        in_specs=[
            pl.BlockSpec((1, _N, 3 + _C), lambda b: (b, 0, 0)),
            pl.BlockSpec((3 + _C, 128), lambda b: (0, 0)),
        ],
        out_specs=pl.BlockSpec((1, _N, 128), lambda b: (b, 0, 0)),
        out_shape=jax.ShapeDtypeStruct((_B, _N, 128), jnp.float32),
    )(pf, w1a)


_NCH = 32
_CW = _N // _NCH
_MAXP = _NCH * _NCH


def _knn_body(xyz_ref, cxT_ref, out_ref, dd_ref, pv_ref, pi_ref):
    b = pl.program_id(0)
    CtT = cxT_ref[0]
    for c in range(_NCH):
        pc = xyz_ref[0, pl.ds(c * _CW, _CW), :]
        pn = jnp.sum(pc * pc, axis=1, keepdims=True)
        dd_ref[pl.ds(c * _CW, _CW), :] = pn - 2.0 * jnp.dot(
            pc, CtT, preferred_element_type=jnp.float32)

    inf = jnp.float32(jnp.inf)
    pv_ref[...] = jnp.full((_MAXP, _TS), inf, jnp.float32)
    pi_ref[...] = jnp.zeros((_MAXP, _TS), jnp.int32)

    sub3 = lax.broadcasted_iota(jnp.int32, (_NCH, _CW, _TS), 1)
    chk3 = lax.broadcasted_iota(jnp.int32, (_NCH, _CW, _TS), 0)
    dd0 = dd_ref[...].reshape(_NCH, _CW, _TS)
    m0 = jnp.min(dd0, axis=1)

    def round_body(carry):
        r, m3, _ = carry
        dd3 = dd_ref[...].reshape(_NCH, _CW, _TS)
        a3 = jnp.min(jnp.where(dd3 == m3[:, None, :], sub3, _CW),
                     axis=1)
        pv_ref[pl.ds(r * _NCH, _NCH), :] = m3
        pi_ref[pl.ds(r * _NCH, _NCH), :] = (
            lax.broadcasted_iota(jnp.int32, (_NCH, _TS), 0) * _CW + a3)
        dd_new = jnp.where(sub3 == a3[:, None, :], inf, dd3)
        dd_ref[...] = dd_new.reshape(_NCH * _CW, _TS)
        m3n = jnp.min(dd_new, axis=1)
        f = jnp.min(m3n, axis=0, keepdims=True)
        cnt = jnp.sum((pv_ref[...] < f).astype(jnp.int32), axis=0,
                      keepdims=True)
        return r + 1, m3n, jnp.all(cnt >= _K)

    def round_cond(carry):
        r, _, done = carry
        return jnp.logical_and(r < _NCH, jnp.logical_not(done))

    rfin, _, _ = lax.while_loop(round_cond, round_body,
                                (jnp.int32(0), m0, jnp.bool_(False)))

    def extract(nrows):
        pv = pv_ref[0:nrows, :]
        pi = pi_ref[0:nrows, :]
        piota = lax.broadcasted_iota(jnp.int32, (nrows, _TS), 0)
        rows = []
        for _ in range(_K):
            mv = jnp.min(pv, axis=0, keepdims=True)
            pos = jnp.min(jnp.where(pv == mv, piota, nrows), axis=0,
                          keepdims=True)
            pmask = piota == pos
            rows.append(jnp.min(jnp.where(pmask, pi, _N), axis=0,
                                keepdims=True))
            pv = jnp.where(pmask, inf, pv)
        return jnp.concatenate(rows, axis=0)

    idxs = lax.cond(rfin <= 8,
                    lambda: extract(8 * _NCH),
                    lambda: extract(_MAXP))
    out_ref[0, 0] = idxs + b * _N


def _run_knn(xyz, new_xyzT):
    nt = _S // _TS
    return pl.pallas_call(
        _knn_body,
        grid=(_B, nt),
        in_specs=[
            pl.BlockSpec((1, _N, 3), lambda b, t: (b, 0, 0)),
            pl.BlockSpec((1, 3, _TS), lambda b, t: (b, 0, t)),
        ],
        out_specs=pl.BlockSpec((1, 1, _K, _TS), lambda b, t: (b, t, 0, 0)),
        out_shape=jax.ShapeDtypeStruct((_B, nt, _K, _TS), jnp.int32),
        scratch_shapes=[
            pltpu.VMEM((_NCH * _CW, _TS), jnp.float32),
            pltpu.VMEM((_MAXP, _TS), jnp.float32),
            pltpu.VMEM((_MAXP, _TS), jnp.int32),
        ],
    )(xyz, new_xyzT)


def _sc_gather_call(table, idx):
    info = plsc.get_sparse_core_info()
    nw = info.num_cores * info.num_subcores
    r = idx.shape[0]
    d = table.shape[1]
    per_w = r // nw
    ch = 128
    n_ch = per_w // ch
    mesh = plsc.VectorSubcoreMesh(core_axis_name="c", subcore_axis_name="s")

    @functools.partial(
        pl.kernel, mesh=mesh,
        out_type=jax.ShapeDtypeStruct((r, d), jnp.float32),
        scratch_types=[
            pltpu.VMEM((ch,), jnp.int32),
            pltpu.VMEM((ch, d), jnp.float32),
            pltpu.SemaphoreType.DMA,
        ],
    )
    def gather_k(table_hbm, idx_hbm, out_hbm, idx_v, rows_v, sem):
        wid = lax.axis_index("s") * info.num_cores + lax.axis_index("c")
        base = wid * per_w

        def step(c, carry):
            off = base + c * ch
            pltpu.sync_copy(idx_hbm.at[pl.ds(off, ch)], idx_v)
            pltpu.async_copy(table_hbm.at[idx_v], rows_v, sem).wait()
            pltpu.sync_copy(rows_v, out_hbm.at[pl.ds(off, ch)])
            return carry

        lax.fori_loop(0, n_ch, step, 0)

    return gather_k(table, idx)


def _mlp_body(g_ref, c_ref, n_ref, w1x_ref, w1n_ref, b1_ref,
              w2_ref, b2_ref, w3_ref, b3_ref, out_ref):
    rows = g_ref[0, 0]
    Ct = c_ref[0]
    Nt = n_ref[0]
    H = (jnp.dot(Nt, w1n_ref[...], preferred_element_type=jnp.float32)
         - jnp.dot(Ct, w1x_ref[...], preferred_element_type=jnp.float32)
         + b1_ref[...])
    pre = rows.reshape(_K, _TM, 128) + H[None, :, :]
    h1 = jnp.maximum(pre, 0.0).reshape(_K * _TM, 128)
    h2 = jnp.maximum(
        jnp.dot(h1, w2_ref[...], preferred_element_type=jnp.float32)
        + b2_ref[...], 0.0)
    h3 = jnp.maximum(
        jnp.dot(h2, w3_ref[...], preferred_element_type=jnp.float32)
        + b3_ref[...], 0.0)
    out_ref[0] = jnp.max(h3.reshape(_K, _TM, 256), axis=0)


def _run_mlp(g4, new_xyz, new_normal, w1x, w1n, b1, W2, b2, W3, b3):
    nt = _S // _TM
    return pl.pallas_call(
        _mlp_body,
        grid=(_B, nt),
        in_specs=[
            pl.BlockSpec((1, 1, _K * _TM, 128), lambda b, t: (b, t, 0, 0)),
            pl.BlockSpec((1, _TM, 3), lambda b, t: (b, t, 0)),
            pl.BlockSpec((1, _TM, 3), lambda b, t: (b, t, 0)),
            pl.BlockSpec((3, 128), lambda b, t: (0, 0)),
            pl.BlockSpec((3, 128), lambda b, t: (0, 0)),
            pl.BlockSpec((1, 128), lambda b, t: (0, 0)),
            pl.BlockSpec((128, 128), lambda b, t: (0, 0)),
            pl.BlockSpec((1, 128), lambda b, t: (0, 0)),
            pl.BlockSpec((128, 256), lambda b, t: (0, 0)),
            pl.BlockSpec((1, 256), lambda b, t: (0, 0)),
        ],
        out_specs=pl.BlockSpec((1, _TM, 256), lambda b, t: (b, t, 0)),
        out_shape=jax.ShapeDtypeStruct((_B, _S, 256), jnp.float32),
    )(g4, new_xyz, new_normal, w1x, w1n, b1, W2, b2, W3, b3)


def kernel(xyz, normal, features, W1, b1, W2, b2, W3, b3):
    xyzT = jnp.transpose(xyz, (0, 2, 1))

    nrmT = jnp.transpose(normal, (0, 2, 1))
    sel = _run_fps(xyzT, nrmT)
    selT = jnp.transpose(sel, (1, 0, 2))
    new_xyz = selT[:, :, 0:3]
    new_normal = selT[:, :, 3:6]

    pf = jnp.concatenate([xyz, features], axis=-1)
    g = _run_g(pf, W1[:3 + _C])

    new_xyzT = jnp.transpose(new_xyz, (0, 2, 1))
    knn_idx = _run_knn(xyz, new_xyzT)
    rows = _sc_gather_call(g.reshape(_B * _N, 128),
                           knn_idx.reshape(-1))
    g4 = rows.reshape(_B, _S // _TM, _K * _TM, 128)

    out = _run_mlp(g4, new_xyz, new_normal,
                   W1[0:3], W1[3 + _C:], b1.reshape(1, 128),
                   W2, b2.reshape(1, 128), W3, b3.reshape(1, 256))
    return new_xyz, new_normal, jnp.transpose(out, (0, 2, 1))


# speedup vs baseline: 1.0773x; 1.0773x over previous
import functools

import jax
import jax.numpy as jnp
from jax import lax
from jax.experimental import pallas as pl
from jax.experimental.pallas import tpu as pltpu
from jax.experimental.pallas import tpu_sc as plsc

_B, _N, _C = 8, 4096, 64
_S = 1024
_K = 32
_TS = 128
_TM = 128


def _fps_body(xyzT_ref, nrmT_ref, out_ref, dist_ref):
    lane = lax.broadcasted_iota(jnp.int32, (_B, _N), 1)
    dist_ref[...] = jnp.full((_B, _N), 1e10, dtype=jnp.float32)

    def step(t, nxt):
        msk = lane == nxt
        X = xyzT_ref[:, 0, :]
        Y = xyzT_ref[:, 1, :]
        Z = xyzT_ref[:, 2, :]
        px = jnp.sum(jnp.where(msk, X, 0.0), axis=1, keepdims=True)
        py = jnp.sum(jnp.where(msk, Y, 0.0), axis=1, keepdims=True)
        pz = jnp.sum(jnp.where(msk, Z, 0.0), axis=1, keepdims=True)
        nx = jnp.sum(jnp.where(msk, nrmT_ref[:, 0, :], 0.0), axis=1, keepdims=True)
        ny = jnp.sum(jnp.where(msk, nrmT_ref[:, 1, :], 0.0), axis=1, keepdims=True)
        nz = jnp.sum(jnp.where(msk, nrmT_ref[:, 2, :], 0.0), axis=1, keepdims=True)
        row = jnp.concatenate([px, py, pz, nx, ny, nz], axis=1)
        out_ref[pl.ds(t, 1)] = row[None]
        d = (X - px) ** 2 + (Y - py) ** 2 + (Z - pz) ** 2
        dist = jnp.minimum(dist_ref[...], d)
        dist_ref[...] = dist
        m = jnp.max(dist, axis=1, keepdims=True)
        return jnp.min(jnp.where(dist == m, lane, _N), axis=1, keepdims=True)

    lax.fori_loop(0, _S, step, jnp.zeros((_B, 1), jnp.int32))


def _run_fps(xyzT, nrmT):
    return pl.pallas_call(
        _fps_body,
        out_shape=jax.ShapeDtypeStruct((_S, _B, 6), jnp.float32),
        scratch_shapes=[pltpu.VMEM((_B, _N), jnp.float32)],
    )(xyzT, nrmT)


def _g_body(pf_ref, w_ref, out_ref):
    out_ref[0] = jnp.dot(pf_ref[0], w_ref[...],
                         preferred_element_type=jnp.float32)


def _run_g(pf, w1a):
    return pl.pallas_call(
        _g_body,
        grid=(_B,),
        in_specs=[
            pl.BlockSpec((1, _N, 3 + _C), lambda b: (b, 0, 0)),
            pl.BlockSpec((3 + _C, 128), lambda b: (0, 0)),
        ],
        out_specs=pl.BlockSpec((1, _N, 128), lambda b: (b, 0, 0)),
        out_shape=jax.ShapeDtypeStruct((_B, _N, 128), jnp.float32),
    )(pf, w1a)


_NCH = 32
_CW = _N // _NCH
_MAXP = _NCH * _NCH


def _knn_body(xyz_ref, cxT_ref, out_ref, dd_ref, pv_ref, pi_ref):
    b = pl.program_id(0)
    CtT = cxT_ref[0]
    for c in range(_NCH):
        pc = xyz_ref[0, pl.ds(c * _CW, _CW), :]
        pn = jnp.sum(pc * pc, axis=1, keepdims=True)
        dd_ref[pl.ds(c * _CW, _CW), :] = pn - 2.0 * jnp.dot(
            pc, CtT, preferred_element_type=jnp.float32)

    inf = jnp.float32(jnp.inf)
    pv_ref[...] = jnp.full((_MAXP, _TS), inf, jnp.float32)
    pi_ref[...] = jnp.zeros((_MAXP, _TS), jnp.int32)

    sub3 = lax.broadcasted_iota(jnp.int32, (_NCH, _CW, _TS), 1)
    chk3 = lax.broadcasted_iota(jnp.int32, (_NCH, _CW, _TS), 0)
    dd0 = dd_ref[...].reshape(_NCH, _CW, _TS)
    m0 = jnp.min(dd0, axis=1)

    def round_body(carry):
        r, m3, _ = carry
        dd3 = dd_ref[...].reshape(_NCH, _CW, _TS)
        a3 = jnp.min(jnp.where(dd3 == m3[:, None, :], sub3, _CW),
                     axis=1)
        pv_ref[pl.ds(r * _NCH, _NCH), :] = m3
        pi_ref[pl.ds(r * _NCH, _NCH), :] = (
            lax.broadcasted_iota(jnp.int32, (_NCH, _TS), 0) * _CW + a3)
        dd_new = jnp.where(sub3 == a3[:, None, :], inf, dd3)
        dd_ref[...] = dd_new.reshape(_NCH * _CW, _TS)
        m3n = jnp.min(dd_new, axis=1)
        f = jnp.min(m3n, axis=0, keepdims=True)
        cnt = jnp.sum((pv_ref[...] < f).astype(jnp.int32), axis=0,
                      keepdims=True)
        return r + 1, m3n, jnp.all(cnt >= _K)

    def round_cond(carry):
        r, _, done = carry
        return jnp.logical_and(r < _NCH, jnp.logical_not(done))

    rfin, _, _ = lax.while_loop(round_cond, round_body,
                                (jnp.int32(0), m0, jnp.bool_(False)))

    def extract(nrows):
        pv = pv_ref[0:nrows, :]
        pi = pi_ref[0:nrows, :]
        piota = lax.broadcasted_iota(jnp.int32, (nrows, _TS), 0)
        rows = []
        for _ in range(_K):
            mv = jnp.min(pv, axis=0, keepdims=True)
            pos = jnp.min(jnp.where(pv == mv, piota, nrows), axis=0,
                          keepdims=True)
            pmask = piota == pos
            rows.append(jnp.min(jnp.where(pmask, pi, _N), axis=0,
                                keepdims=True))
            pv = jnp.where(pmask, inf, pv)
        return jnp.concatenate(rows, axis=0)

    idxs = lax.cond(rfin <= 8,
                    lambda: extract(8 * _NCH),
                    lambda: extract(_MAXP))
    out_ref[0, 0] = idxs + b * _N


def _run_knn(xyz, new_xyzT):
    nt = _S // _TS
    return pl.pallas_call(
        _knn_body,
        grid=(_B, nt),
        in_specs=[
            pl.BlockSpec((1, _N, 3), lambda b, t: (b, 0, 0)),
            pl.BlockSpec((1, 3, _TS), lambda b, t: (b, 0, t)),
        ],
        out_specs=pl.BlockSpec((1, 1, _K, _TS), lambda b, t: (b, t, 0, 0)),
        out_shape=jax.ShapeDtypeStruct((_B, nt, _K, _TS), jnp.int32),
        scratch_shapes=[
            pltpu.VMEM((_NCH * _CW, _TS), jnp.float32),
            pltpu.VMEM((_MAXP, _TS), jnp.float32),
            pltpu.VMEM((_MAXP, _TS), jnp.int32),
        ],
    )(xyz, new_xyzT)


def _sc_gather_call(table, idx):
    info = plsc.get_sparse_core_info()
    nw = info.num_cores * info.num_subcores
    r = idx.shape[0]
    d = table.shape[1]
    per_w = r // nw
    ch = 128
    n_ch = per_w // ch
    mesh = plsc.VectorSubcoreMesh(core_axis_name="c", subcore_axis_name="s")

    @functools.partial(
        pl.kernel, mesh=mesh,
        out_type=jax.ShapeDtypeStruct((r, d), jnp.float32),
        scratch_types=[
            pltpu.VMEM((per_w,), jnp.int32),
            pltpu.VMEM((ch, d), jnp.float32),
            pltpu.VMEM((ch, d), jnp.float32),
            pltpu.SemaphoreType.DMA,
        ],
    )
    def gather_k(table_hbm, idx_hbm, out_hbm, idx_v, rows0_v, rows1_v, sem):
        wid = lax.axis_index("s") * info.num_cores + lax.axis_index("c")
        base = wid * per_w
        pltpu.sync_copy(idx_hbm.at[pl.ds(base, per_w)], idx_v)

        def pair(j, carry):
            c0 = 2 * j
            cp0 = pltpu.async_copy(
                table_hbm.at[idx_v.at[pl.ds(c0 * ch, ch)]], rows0_v, sem)
            cp1 = pltpu.async_copy(
                table_hbm.at[idx_v.at[pl.ds((c0 + 1) * ch, ch)]], rows1_v, sem)
            cp0.wait()
            pltpu.sync_copy(rows0_v, out_hbm.at[pl.ds(base + c0 * ch, ch)])
            cp1.wait()
            pltpu.sync_copy(rows1_v,
                            out_hbm.at[pl.ds(base + (c0 + 1) * ch, ch)])
            return carry

        lax.fori_loop(0, n_ch // 2, pair, 0)

    return gather_k(table, idx)


def _mlp_body(g_ref, c_ref, n_ref, w1x_ref, w1n_ref, b1_ref,
              w2_ref, b2_ref, w3_ref, b3_ref, out_ref):
    rows = g_ref[0, 0]
    Ct = c_ref[0]
    Nt = n_ref[0]
    H = (jnp.dot(Nt, w1n_ref[...], preferred_element_type=jnp.float32)
         - jnp.dot(Ct, w1x_ref[...], preferred_element_type=jnp.float32)
         + b1_ref[...])
    pre = rows.reshape(_K, _TM, 128) + H[None, :, :]
    h1 = jnp.maximum(pre, 0.0).reshape(_K * _TM, 128)
    h2 = jnp.maximum(
        jnp.dot(h1, w2_ref[...], preferred_element_type=jnp.float32)
        + b2_ref[...], 0.0)
    h3 = jnp.maximum(
        jnp.dot(h2, w3_ref[...], preferred_element_type=jnp.float32)
        + b3_ref[...], 0.0)
    out_ref[0] = jnp.max(h3.reshape(_K, _TM, 256), axis=0)


def _run_mlp(g4, new_xyz, new_normal, w1x, w1n, b1, W2, b2, W3, b3):
    nt = _S // _TM
    return pl.pallas_call(
        _mlp_body,
        grid=(_B, nt),
        in_specs=[
            pl.BlockSpec((1, 1, _K * _TM, 128), lambda b, t: (b, t, 0, 0)),
            pl.BlockSpec((1, _TM, 3), lambda b, t: (b, t, 0)),
            pl.BlockSpec((1, _TM, 3), lambda b, t: (b, t, 0)),
            pl.BlockSpec((3, 128), lambda b, t: (0, 0)),
            pl.BlockSpec((3, 128), lambda b, t: (0, 0)),
            pl.BlockSpec((1, 128), lambda b, t: (0, 0)),
            pl.BlockSpec((128, 128), lambda b, t: (0, 0)),
            pl.BlockSpec((1, 128), lambda b, t: (0, 0)),
            pl.BlockSpec((128, 256), lambda b, t: (0, 0)),
            pl.BlockSpec((1, 256), lambda b, t: (0, 0)),
        ],
        out_specs=pl.BlockSpec((1, _TM, 256), lambda b, t: (b, t, 0)),
        out_shape=jax.ShapeDtypeStruct((_B, _S, 256), jnp.float32),
    )(g4, new_xyz, new_normal, w1x, w1n, b1, W2, b2, W3, b3)


def kernel(xyz, normal, features, W1, b1, W2, b2, W3, b3):
    xyzT = jnp.transpose(xyz, (0, 2, 1))

    nrmT = jnp.transpose(normal, (0, 2, 1))
    sel = _run_fps(xyzT, nrmT)
    selT = jnp.transpose(sel, (1, 0, 2))
    new_xyz = selT[:, :, 0:3]
    new_normal = selT[:, :, 3:6]

    pf = jnp.concatenate([xyz, features], axis=-1)
    g = _run_g(pf, W1[:3 + _C])

    new_xyzT = jnp.transpose(new_xyz, (0, 2, 1))
    knn_idx = _run_knn(xyz, new_xyzT)
    rows = _sc_gather_call(g.reshape(_B * _N, 128),
                           knn_idx.reshape(-1))
    g4 = rows.reshape(_B, _S // _TM, _K * _TM, 128)

    out = _run_mlp(g4, new_xyz, new_normal,
                   W1[0:3], W1[3 + _C:], b1.reshape(1, 128),
                   W2, b2.reshape(1, 128), W3, b3.reshape(1, 256))
    return new_xyz, new_normal, jnp.transpose(out, (0, 2, 1))


# speedup vs baseline: 1.0857x; 1.0078x over previous
import functools

import jax
import jax.numpy as jnp
from jax import lax
from jax.experimental import pallas as pl
from jax.experimental.pallas import tpu as pltpu
from jax.experimental.pallas import tpu_sc as plsc

_B, _N, _C = 8, 4096, 64
_S = 1024
_K = 32
_TS = 128
_TM = 128


def _fps_body(xyzT_ref, nrmT_ref, out_ref, dist_ref):
    lane = lax.broadcasted_iota(jnp.int32, (_B, _N), 1)
    dist_ref[...] = jnp.full((_B, _N), 1e10, dtype=jnp.float32)

    def step(t, nxt):
        msk = lane == nxt
        X = xyzT_ref[:, 0, :]
        Y = xyzT_ref[:, 1, :]
        Z = xyzT_ref[:, 2, :]
        px = jnp.sum(jnp.where(msk, X, 0.0), axis=1, keepdims=True)
        py = jnp.sum(jnp.where(msk, Y, 0.0), axis=1, keepdims=True)
        pz = jnp.sum(jnp.where(msk, Z, 0.0), axis=1, keepdims=True)
        nx = jnp.sum(jnp.where(msk, nrmT_ref[:, 0, :], 0.0), axis=1, keepdims=True)
        ny = jnp.sum(jnp.where(msk, nrmT_ref[:, 1, :], 0.0), axis=1, keepdims=True)
        nz = jnp.sum(jnp.where(msk, nrmT_ref[:, 2, :], 0.0), axis=1, keepdims=True)
        row = jnp.concatenate([px, py, pz, nx, ny, nz], axis=1)
        out_ref[pl.ds(t, 1)] = row[None]
        d = (X - px) ** 2 + (Y - py) ** 2 + (Z - pz) ** 2
        dist = jnp.minimum(dist_ref[...], d)
        dist_ref[...] = dist
        m = jnp.max(dist, axis=1, keepdims=True)
        return jnp.min(jnp.where(dist == m, lane, _N), axis=1, keepdims=True)

    lax.fori_loop(0, _S, step, jnp.zeros((_B, 1), jnp.int32))


def _run_fps(xyzT, nrmT):
    return pl.pallas_call(
        _fps_body,
        out_shape=jax.ShapeDtypeStruct((_S, _B, 6), jnp.float32),
        scratch_shapes=[pltpu.VMEM((_B, _N), jnp.float32)],
    )(xyzT, nrmT)


def _g_body(xyz_ref, f_ref, wx_ref, wf_ref, out_ref):
    out_ref[0] = (jnp.dot(xyz_ref[0], wx_ref[...],
                          preferred_element_type=jnp.float32)
                  + jnp.dot(f_ref[0], wf_ref[...],
                            preferred_element_type=jnp.float32))


def _run_g(xyz, features, w1x, w1f):
    return pl.pallas_call(
        _g_body,
        grid=(_B,),
        in_specs=[
            pl.BlockSpec((1, _N, 3), lambda b: (b, 0, 0)),
            pl.BlockSpec((1, _N, _C), lambda b: (b, 0, 0)),
            pl.BlockSpec((3, 128), lambda b: (0, 0)),
            pl.BlockSpec((_C, 128), lambda b: (0, 0)),
        ],
        out_specs=pl.BlockSpec((1, _N, 128), lambda b: (b, 0, 0)),
        out_shape=jax.ShapeDtypeStruct((_B, _N, 128), jnp.float32),
    )(xyz, features, w1x, w1f)


_NCH = 32
_CW = _N // _NCH
_MAXP = _NCH * _NCH


def _knn_body(xyz_ref, cxT_ref, out_ref, dd_ref, pv_ref, pi_ref):
    b = pl.program_id(0)
    CtT = cxT_ref[0]
    for c in range(_NCH):
        pc = xyz_ref[0, pl.ds(c * _CW, _CW), :]
        pn = jnp.sum(pc * pc, axis=1, keepdims=True)
        dd_ref[pl.ds(c * _CW, _CW), :] = pn - 2.0 * jnp.dot(
            pc, CtT, preferred_element_type=jnp.float32)

    inf = jnp.float32(jnp.inf)
    pv_ref[...] = jnp.full((_MAXP, _TS), inf, jnp.float32)
    pi_ref[...] = jnp.zeros((_MAXP, _TS), jnp.int32)

    sub3 = lax.broadcasted_iota(jnp.int32, (_NCH, _CW, _TS), 1)
    chk3 = lax.broadcasted_iota(jnp.int32, (_NCH, _CW, _TS), 0)
    dd0 = dd_ref[...].reshape(_NCH, _CW, _TS)
    m0 = jnp.min(dd0, axis=1)

    def round_body(carry):
        r, m3, _ = carry
        dd3 = dd_ref[...].reshape(_NCH, _CW, _TS)
        a3 = jnp.min(jnp.where(dd3 == m3[:, None, :], sub3, _CW),
                     axis=1)
        pv_ref[pl.ds(r * _NCH, _NCH), :] = m3
        pi_ref[pl.ds(r * _NCH, _NCH), :] = (
            lax.broadcasted_iota(jnp.int32, (_NCH, _TS), 0) * _CW + a3)
        dd_new = jnp.where(sub3 == a3[:, None, :], inf, dd3)
        dd_ref[...] = dd_new.reshape(_NCH * _CW, _TS)
        m3n = jnp.min(dd_new, axis=1)
        f = jnp.min(m3n, axis=0, keepdims=True)
        cnt = jnp.sum((pv_ref[...] < f).astype(jnp.int32), axis=0,
                      keepdims=True)
        return r + 1, m3n, jnp.all(cnt >= _K)

    def round_cond(carry):
        r, _, done = carry
        return jnp.logical_and(r < _NCH, jnp.logical_not(done))

    rfin, _, _ = lax.while_loop(round_cond, round_body,
                                (jnp.int32(0), m0, jnp.bool_(False)))

    def extract(nrows):
        pv = pv_ref[0:nrows, :]
        pi = pi_ref[0:nrows, :]
        piota = lax.broadcasted_iota(jnp.int32, (nrows, _TS), 0)
        rows = []
        for _ in range(_K):
            mv = jnp.min(pv, axis=0, keepdims=True)
            pos = jnp.min(jnp.where(pv == mv, piota, nrows), axis=0,
                          keepdims=True)
            pmask = piota == pos
            rows.append(jnp.min(jnp.where(pmask, pi, _N), axis=0,
                                keepdims=True))
            pv = jnp.where(pmask, inf, pv)
        return jnp.concatenate(rows, axis=0)

    idxs = lax.cond(rfin <= 8,
                    lambda: extract(8 * _NCH),
                    lambda: extract(_MAXP))
    out_ref[0, 0] = idxs + b * _N


def _run_knn(xyz, new_xyzT):
    nt = _S // _TS
    return pl.pallas_call(
        _knn_body,
        grid=(_B, nt),
        in_specs=[
            pl.BlockSpec((1, _N, 3), lambda b, t: (b, 0, 0)),
            pl.BlockSpec((1, 3, _TS), lambda b, t: (b, 0, t)),
        ],
        out_specs=pl.BlockSpec((1, 1, _K, _TS), lambda b, t: (b, t, 0, 0)),
        out_shape=jax.ShapeDtypeStruct((_B, nt, _K, _TS), jnp.int32),
        scratch_shapes=[
            pltpu.VMEM((_NCH * _CW, _TS), jnp.float32),
            pltpu.VMEM((_MAXP, _TS), jnp.float32),
            pltpu.VMEM((_MAXP, _TS), jnp.int32),
        ],
    )(xyz, new_xyzT)


def _sc_gather_call(table, idx):
    info = plsc.get_sparse_core_info()
    nw = info.num_cores * info.num_subcores
    r = idx.shape[0]
    d = table.shape[1]
    per_w = r // nw
    ch = 128
    n_ch = per_w // ch
    mesh = plsc.VectorSubcoreMesh(core_axis_name="c", subcore_axis_name="s")

    @functools.partial(
        pl.kernel, mesh=mesh,
        out_type=jax.ShapeDtypeStruct((r, d), jnp.float32),
        scratch_types=[
            pltpu.VMEM((per_w,), jnp.int32),
            pltpu.VMEM((ch, d), jnp.float32),
            pltpu.VMEM((ch, d), jnp.float32),
            pltpu.SemaphoreType.DMA,
        ],
    )
    def gather_k(table_hbm, idx_hbm, out_hbm, idx_v, rows0_v, rows1_v, sem):
        wid = lax.axis_index("s") * info.num_cores + lax.axis_index("c")
        base = wid * per_w
        pltpu.sync_copy(idx_hbm.at[pl.ds(base, per_w)], idx_v)

        def pair(j, carry):
            c0 = 2 * j
            cp0 = pltpu.async_copy(
                table_hbm.at[idx_v.at[pl.ds(c0 * ch, ch)]], rows0_v, sem)
            cp1 = pltpu.async_copy(
                table_hbm.at[idx_v.at[pl.ds((c0 + 1) * ch, ch)]], rows1_v, sem)
            cp0.wait()
            pltpu.sync_copy(rows0_v, out_hbm.at[pl.ds(base + c0 * ch, ch)])
            cp1.wait()
            pltpu.sync_copy(rows1_v,
                            out_hbm.at[pl.ds(base + (c0 + 1) * ch, ch)])
            return carry

        lax.fori_loop(0, n_ch // 2, pair, 0)

    return gather_k(table, idx)


def _mlp_body(g_ref, c_ref, n_ref, w1x_ref, w1n_ref, b1_ref,
              w2_ref, b2_ref, w3_ref, b3_ref, out_ref):
    rows = g_ref[0, 0]
    Ct = c_ref[0]
    Nt = n_ref[0]
    H = (jnp.dot(Nt, w1n_ref[...], preferred_element_type=jnp.float32)
         - jnp.dot(Ct, w1x_ref[...], preferred_element_type=jnp.float32)
         + b1_ref[...])
    pre = rows.reshape(_K, _TM, 128) + H[None, :, :]
    h1 = jnp.maximum(pre, 0.0).reshape(_K * _TM, 128)
    h2 = jnp.maximum(
        jnp.dot(h1, w2_ref[...], preferred_element_type=jnp.float32)
        + b2_ref[...], 0.0)
    h3 = jnp.maximum(
        jnp.dot(h2, w3_ref[...], preferred_element_type=jnp.float32)
        + b3_ref[...], 0.0)
    out_ref[0] = jnp.max(h3.reshape(_K, _TM, 256), axis=0).T


def _run_mlp(g4, new_xyz, new_normal, w1x, w1n, b1, W2, b2, W3, b3):
    nt = _S // _TM
    return pl.pallas_call(
        _mlp_body,
        grid=(_B, nt),
        in_specs=[
            pl.BlockSpec((1, 1, _K * _TM, 128), lambda b, t: (b, t, 0, 0)),
            pl.BlockSpec((1, _TM, 3), lambda b, t: (b, t, 0)),
            pl.BlockSpec((1, _TM, 3), lambda b, t: (b, t, 0)),
            pl.BlockSpec((3, 128), lambda b, t: (0, 0)),
            pl.BlockSpec((3, 128), lambda b, t: (0, 0)),
            pl.BlockSpec((1, 128), lambda b, t: (0, 0)),
            pl.BlockSpec((128, 128), lambda b, t: (0, 0)),
            pl.BlockSpec((1, 128), lambda b, t: (0, 0)),
            pl.BlockSpec((128, 256), lambda b, t: (0, 0)),
            pl.BlockSpec((1, 256), lambda b, t: (0, 0)),
        ],
        out_specs=pl.BlockSpec((1, 256, _TM), lambda b, t: (b, 0, t)),
        out_shape=jax.ShapeDtypeStruct((_B, 256, _S), jnp.float32),
    )(g4, new_xyz, new_normal, w1x, w1n, b1, W2, b2, W3, b3)


def kernel(xyz, normal, features, W1, b1, W2, b2, W3, b3):
    xyzT = jnp.transpose(xyz, (0, 2, 1))

    nrmT = jnp.transpose(normal, (0, 2, 1))
    sel = _run_fps(xyzT, nrmT)
    selT = jnp.transpose(sel, (1, 0, 2))
    new_xyz = selT[:, :, 0:3]
    new_normal = selT[:, :, 3:6]

    g = _run_g(xyz, features, W1[0:3], W1[3:3 + _C])

    new_xyzT = jnp.transpose(new_xyz, (0, 2, 1))
    knn_idx = _run_knn(xyz, new_xyzT)
    rows = _sc_gather_call(g.reshape(_B * _N, 128),
                           knn_idx.reshape(-1))
    g4 = rows.reshape(_B, _S // _TM, _K * _TM, 128)

    out = _run_mlp(g4, new_xyz, new_normal,
                   W1[0:3], W1[3 + _C:], b1.reshape(1, 128),
                   W2, b2.reshape(1, 128), W3, b3.reshape(1, 256))
    return new_xyz, new_normal, out
